# Initial kernel scaffold; baseline (speedup 1.0000x reference)
import jax
import jax.numpy as jnp
from jax.experimental import pallas as pl


def kernel(node_feats, edge_index, W_msg, b_msg, W_upd, b_upd, W_out1, b_out1, W_out2, b_out2):
    raise NotImplementedError("write your pallas kernel here")


# speedup vs baseline: 11.0137x; 11.0137x over previous
import functools

import jax
import jax.numpy as jnp
from jax import lax
from jax.experimental import pallas as pl
from jax.experimental.pallas import tpu as pltpu
from jax.experimental.pallas import tpu_sc as plsc

EPS = 1e-05
B, N, DIN, DN = 8, 1250, 128, 128
E = 320000
H_OUT = 256
N_CLASSES = 10
BN = B * N

NC, NS = 2, 16
NW = NC * NS
EPT = E // NW
CH = 125
NCH = EPT // CH
WB_TILES = 10
WB_ROWS = BN // WB_TILES
DEGW = 16


G = 8
NG = NCH // G


def _make_sc_seg():
  mesh = plsc.VectorSubcoreMesh(core_axis_name="c", subcore_axis_name="s")

  def body(m_hbm, srcr, dstr, zerosr, agg_out,
           src_b0, src_b1, dst_b0, dst_b1, rows0, rows1,
           agg_sh, semr0, semr1, semi0, semi1):
    c = lax.axis_index("c")
    s = lax.axis_index("s")
    wid = c * NS + s
    r0 = s * WB_ROWS
    rows = (rows0, rows1)
    semr = (semr0, semr1)

    @pl.when(s < WB_TILES)
    def _zero():
      pltpu.sync_copy(zerosr.at[pl.ds(r0, WB_ROWS)],
                      agg_sh.at[pl.ds(r0, WB_ROWS)])

    def idx_start(g, src_b, dst_b, semi):
      off = pl.multiple_of(g * G, G)
      pltpu.async_copy(srcr.at[wid, pl.ds(off, G)], src_b, semi)
      pltpu.async_copy(dstr.at[wid, pl.ds(off, G)], dst_b, semi)

    def idx_wait(g, src_b, dst_b, semi):
      off = pl.multiple_of(g * G, G)
      pltpu.make_async_copy(srcr.at[wid, pl.ds(off, G)], src_b, semi).wait()
      pltpu.make_async_copy(dstr.at[wid, pl.ds(off, G)], dst_b, semi).wait()

    def process_group(src_b, dst_b):
      cps = []
      for j in range(G):
        cps.append(pltpu.async_copy(m_hbm.at[src_b.at[j]], rows[j % 2],
                                    semr[j % 2]))
        if j > 0:
          cps[j - 1].wait()
          pltpu.sync_copy(rows[(j - 1) % 2], agg_sh.at[dst_b.at[j - 1]],
                          add=True)
      cps[G - 1].wait()
      pltpu.sync_copy(rows[(G - 1) % 2], agg_sh.at[dst_b.at[G - 1]], add=True)

    idx_start(0, src_b0, dst_b0, semi0)
    plsc.subcore_barrier()

    def pair(k, carry):
      g0 = 2 * k
      idx_start(g0 + 1, src_b1, dst_b1, semi1)
      idx_wait(g0, src_b0, dst_b0, semi0)
      process_group(src_b0, dst_b0)

      @pl.when(k < NG // 2 - 1)
      def _prefetch():
        idx_start(g0 + 2, src_b0, dst_b0, semi0)

      idx_wait(g0 + 1, src_b1, dst_b1, semi1)
      process_group(src_b1, dst_b1)
      return carry

    lax.fori_loop(0, NG // 2, pair, 0)
    plsc.subcore_barrier()

    @pl.when(s < WB_TILES)
    def _writeback():
      pltpu.sync_copy(agg_sh.at[pl.ds(r0, WB_ROWS)],
                      agg_out.at[c, pl.ds(r0, WB_ROWS)])

  scratch = [
      pltpu.VMEM((G, CH), jnp.int32),
      pltpu.VMEM((G, CH), jnp.int32),
      pltpu.VMEM((G, CH), jnp.int32),
      pltpu.VMEM((G, CH), jnp.int32),
      pltpu.VMEM((CH, DN), jnp.float32),
      pltpu.VMEM((CH, DN), jnp.float32),
      pltpu.VMEM_SHARED((BN, DN), jnp.float32),
      pltpu.SemaphoreType.DMA,
      pltpu.SemaphoreType.DMA,
      pltpu.SemaphoreType.DMA,
      pltpu.SemaphoreType.DMA,
  ]
  return pl.kernel(
      body, (jax.ShapeDtypeStruct((NC, BN, DN), jnp.float32),),
      mesh=mesh, scratch_types=tuple(scratch))


def _make_sc_deg():
  mesh = plsc.VectorSubcoreMesh(core_axis_name="c", subcore_axis_name="s")

  def body(dstr, zerosdr, onesr, deg_out,
           dst_b0, dst_b1, ones_v, deg_sh, semi0, semi1):
    c = lax.axis_index("c")
    s = lax.axis_index("s")
    wid = c * NS + s
    r0 = s * WB_ROWS

    @pl.when(s < WB_TILES)
    def _zero():
      pltpu.sync_copy(zerosdr.at[pl.ds(r0, WB_ROWS)],
                      deg_sh.at[pl.ds(r0, WB_ROWS)])

    pltpu.sync_copy(onesr, ones_v)

    def idx_start(g, dst_b, semi):
      off = pl.multiple_of(g * G, G)
      pltpu.async_copy(dstr.at[wid, pl.ds(off, G)], dst_b, semi)

    def idx_wait(g, dst_b, semi):
      off = pl.multiple_of(g * G, G)
      pltpu.make_async_copy(dstr.at[wid, pl.ds(off, G)], dst_b, semi).wait()

    def process_group(dst_b):
      for j in range(G):
        pltpu.sync_copy(ones_v, deg_sh.at[dst_b.at[j]], add=True)

    idx_start(0, dst_b0, semi0)
    plsc.subcore_barrier()

    def pair(k, carry):
      g0 = 2 * k
      idx_start(g0 + 1, dst_b1, semi1)
      idx_wait(g0, dst_b0, semi0)
      process_group(dst_b0)

      @pl.when(k < NG // 2 - 1)
      def _prefetch():
        idx_start(g0 + 2, dst_b0, semi0)

      idx_wait(g0 + 1, dst_b1, semi1)
      process_group(dst_b1)
      return carry

    lax.fori_loop(0, NG // 2, pair, 0)
    plsc.subcore_barrier()

    @pl.when(s < WB_TILES)
    def _writeback():
      pltpu.sync_copy(deg_sh.at[pl.ds(r0, WB_ROWS)],
                      deg_out.at[c, pl.ds(r0, WB_ROWS)])

  scratch = [
      pltpu.VMEM((G, CH), jnp.int32),
      pltpu.VMEM((G, CH), jnp.int32),
      pltpu.VMEM((CH, DEGW), jnp.float32),
      pltpu.VMEM_SHARED((BN, DEGW), jnp.float32),
      pltpu.SemaphoreType.DMA,
      pltpu.SemaphoreType.DMA,
  ]
  return pl.kernel(
      body, (jax.ShapeDtypeStruct((NC, BN, DEGW), jnp.float32),),
      mesh=mesh, scratch_types=tuple(scratch))


_sc_seg = _make_sc_seg()
_sc_deg = _make_sc_deg()

RB = 2000


def _msg_body(h_ref, w_ref, b_ref, o_ref):
  o_ref[...] = jnp.maximum(
      jnp.dot(h_ref[...], w_ref[...], preferred_element_type=jnp.float32)
      + b_ref[...], 0.0)


_msg = pl.pallas_call(
    _msg_body,
    grid=(BN // RB,),
    in_specs=[
        pl.BlockSpec((RB, DN), lambda i: (i, 0)),
        pl.BlockSpec((DN, DN), lambda i: (0, 0)),
        pl.BlockSpec((1, DN), lambda i: (0, 0)),
    ],
    out_specs=pl.BlockSpec((RB, DN), lambda i: (i, 0)),
    out_shape=jax.ShapeDtypeStruct((BN, DN), jnp.float32),
)


def _upd_body(h_ref, a_ref, d_ref, wua_ref, wub_ref, bu_ref, wm_ref, bm_ref,
              h1_ref, m2_ref):
  agg = a_ref[0] + a_ref[1]
  deg = d_ref[0, :, 0:1] + d_ref[1, :, 0:1]
  mask = jnp.where(deg > EPS, 1.0, 0.0)
  x = jnp.dot(h_ref[...], wua_ref[...], preferred_element_type=jnp.float32)
  x = x + jnp.dot(agg, wub_ref[...], preferred_element_type=jnp.float32)
  h1 = jnp.maximum(x + bu_ref[...], 0.0) * mask
  h1_ref[...] = h1
  m2_ref[...] = jnp.maximum(
      jnp.dot(h1, wm_ref[...], preferred_element_type=jnp.float32)
      + bm_ref[...], 0.0)


_upd = pl.pallas_call(
    _upd_body,
    grid=(BN // RB,),
    in_specs=[
        pl.BlockSpec((RB, DN), lambda i: (i, 0)),
        pl.BlockSpec((NC, RB, DN), lambda i: (0, i, 0)),
        pl.BlockSpec((NC, RB, DEGW), lambda i: (0, i, 0)),
        pl.BlockSpec((DN, DN), lambda i: (0, 0)),
        pl.BlockSpec((DN, DN), lambda i: (0, 0)),
        pl.BlockSpec((1, DN), lambda i: (0, 0)),
        pl.BlockSpec((DN, DN), lambda i: (0, 0)),
        pl.BlockSpec((1, DN), lambda i: (0, 0)),
    ],
    out_specs=[
        pl.BlockSpec((RB, DN), lambda i: (i, 0)),
        pl.BlockSpec((RB, DN), lambda i: (i, 0)),
    ],
    out_shape=[
        jax.ShapeDtypeStruct((BN, DN), jnp.float32),
        jax.ShapeDtypeStruct((BN, DN), jnp.float32),
    ],
)


def _tail_body(h_ref, a_ref, d_ref, wua_ref, wub_ref, bu_ref, w1_ref, b1_ref,
               w2_ref, b2_ref, o_ref):
  agg = a_ref[0] + a_ref[1]
  deg = d_ref[0, :, 0:1] + d_ref[1, :, 0:1]
  mask = jnp.where(deg > EPS, 1.0, 0.0)
  x = jnp.dot(h_ref[...], wua_ref[...], preferred_element_type=jnp.float32)
  x = x + jnp.dot(agg, wub_ref[...], preferred_element_type=jnp.float32)
  h2 = jnp.maximum(x + bu_ref[...], 0.0) * mask
  gid = lax.broadcasted_iota(jnp.int32, (B, BN), 1) // N
  bid = lax.broadcasted_iota(jnp.int32, (B, BN), 0)
  sel = jnp.where(gid == bid, 1.0, 0.0)
  pooled = jnp.dot(sel, h2, preferred_element_type=jnp.float32)
  counts = jnp.dot(sel, mask, preferred_element_type=jnp.float32)
  combined = pooled / counts
  hidden = jnp.maximum(
      jnp.dot(combined, w1_ref[...], preferred_element_type=jnp.float32)
      + b1_ref[...], 0.0)
  o_ref[...] = jnp.dot(hidden, w2_ref[...],
                       preferred_element_type=jnp.float32) + b2_ref[...]


_tail = pl.pallas_call(
    _tail_body,
    out_shape=jax.ShapeDtypeStruct((B, DN), jnp.float32),
)


@jax.jit
def kernel(node_feats, edge_index, W_msg, b_msg, W_upd, b_upd, W_out1, b_out1,
           W_out2, b_out2):
  h0 = node_feats.reshape(BN, DIN)
  ei = edge_index.astype(jnp.int32)
  src3 = ei[0].reshape(NW, NCH, CH)
  dst3 = ei[1].reshape(NW, NCH, CH)
  zeros = jnp.zeros((BN, DN), jnp.float32)
  zerosd = jnp.zeros((BN, DEGW), jnp.float32)
  ones = jnp.ones((CH, DEGW), jnp.float32)
  Wua = W_upd[:DN]
  Wub = W_upd[DN:]
  bm = b_msg.reshape(1, DN)
  bu = b_upd.reshape(1, DN)
  b1 = b_out1.reshape(1, H_OUT)
  W2p = jnp.pad(W_out2, ((0, 0), (0, DN - N_CLASSES)))
  b2p = jnp.pad(b_out2, (0, DN - N_CLASSES)).reshape(1, DN)

  m1 = _msg(h0, W_msg, bm)
  (degP,) = _sc_deg(dst3, zerosd, ones)
  (aggP1,) = _sc_seg(m1, src3, dst3, zeros)
  h1, m2 = _upd(h0, aggP1, degP, Wua, Wub, bu, W_msg, bm)
  (aggP2,) = _sc_seg(m2, src3, dst3, zeros)
  logits_p = _tail(h1, aggP2, degP, Wua, Wub, bu, W_out1, b1, W2p, b2p)
  return logits_p[:, :N_CLASSES]


# speedup vs baseline: 11.7085x; 1.0631x over previous
import functools

import jax
import jax.numpy as jnp
from jax import lax
from jax.experimental import pallas as pl
from jax.experimental.pallas import tpu as pltpu
from jax.experimental.pallas import tpu_sc as plsc

EPS = 1e-05
B, N, DIN, DN = 8, 1250, 128, 128
E = 320000
H_OUT = 256
N_CLASSES = 10
BN = B * N

NC, NS = 2, 16
NW = NC * NS
EPT = E // NW
CH = 125
NCH = EPT // CH
WB_TILES = 10
WB_ROWS = BN // WB_TILES
DEGW = 16


G = 8
NG = NCH // G


def _make_sc_seg():
  mesh = plsc.VectorSubcoreMesh(core_axis_name="c", subcore_axis_name="s")

  def body(m_hbm, edger, zerosr, agg_out,
           src_b0, src_b1, dst_b0, dst_b1, rows0, rows1,
           agg_sh, semr0, semr1, semi0, semi1):
    c = lax.axis_index("c")
    s = lax.axis_index("s")
    wid = c * NS + s
    r0 = s * WB_ROWS
    rows = (rows0, rows1)
    semr = (semr0, semr1)

    @pl.when(s < WB_TILES)
    def _zero():
      pltpu.sync_copy(zerosr, agg_sh.at[pl.ds(r0, WB_ROWS)])

    def idx_start(g, src_b, dst_b, semi):
      off = pl.multiple_of(g * G, G)
      pltpu.async_copy(edger.at[0, wid, pl.ds(off, G)], src_b, semi)
      pltpu.async_copy(edger.at[1, wid, pl.ds(off, G)], dst_b, semi)

    def idx_wait(g, src_b, dst_b, semi):
      off = pl.multiple_of(g * G, G)
      pltpu.make_async_copy(edger.at[0, wid, pl.ds(off, G)], src_b, semi).wait()
      pltpu.make_async_copy(edger.at[1, wid, pl.ds(off, G)], dst_b, semi).wait()

    def process_group(src_b, dst_b):
      cps = []
      for j in range(G):
        cps.append(pltpu.async_copy(m_hbm.at[src_b.at[j]], rows[j % 2],
                                    semr[j % 2]))
        if j > 0:
          cps[j - 1].wait()
          pltpu.sync_copy(rows[(j - 1) % 2], agg_sh.at[dst_b.at[j - 1]],
                          add=True)
      cps[G - 1].wait()
      pltpu.sync_copy(rows[(G - 1) % 2], agg_sh.at[dst_b.at[G - 1]], add=True)

    idx_start(0, src_b0, dst_b0, semi0)
    plsc.subcore_barrier()

    def pair(k, carry):
      g0 = 2 * k
      idx_start(g0 + 1, src_b1, dst_b1, semi1)
      idx_wait(g0, src_b0, dst_b0, semi0)
      process_group(src_b0, dst_b0)

      @pl.when(k < NG // 2 - 1)
      def _prefetch():
        idx_start(g0 + 2, src_b0, dst_b0, semi0)

      idx_wait(g0 + 1, src_b1, dst_b1, semi1)
      process_group(src_b1, dst_b1)
      return carry

    lax.fori_loop(0, NG // 2, pair, 0)
    plsc.subcore_barrier()

    @pl.when(s < WB_TILES)
    def _writeback():
      pltpu.sync_copy(agg_sh.at[pl.ds(r0, WB_ROWS)],
                      agg_out.at[c, pl.ds(r0, WB_ROWS)])

  scratch = [
      pltpu.VMEM((G, CH), jnp.int32),
      pltpu.VMEM((G, CH), jnp.int32),
      pltpu.VMEM((G, CH), jnp.int32),
      pltpu.VMEM((G, CH), jnp.int32),
      pltpu.VMEM((CH, DN), jnp.float32),
      pltpu.VMEM((CH, DN), jnp.float32),
      pltpu.VMEM_SHARED((BN, DN), jnp.float32),
      pltpu.SemaphoreType.DMA,
      pltpu.SemaphoreType.DMA,
      pltpu.SemaphoreType.DMA,
      pltpu.SemaphoreType.DMA,
  ]
  return pl.kernel(
      body, (jax.ShapeDtypeStruct((NC, BN, DN), jnp.float32),),
      mesh=mesh, scratch_types=tuple(scratch))


def _make_sc_deg():
  mesh = plsc.VectorSubcoreMesh(core_axis_name="c", subcore_axis_name="s")

  def body(edger, zerosdr, onesr, deg_out,
           dst_b0, dst_b1, ones_v, deg_sh, sems, semi0, semi1):
    c = lax.axis_index("c")
    s = lax.axis_index("s")
    wid = c * NS + s
    r0 = s * WB_ROWS

    @pl.when(s < WB_TILES)
    def _zero():
      pltpu.sync_copy(zerosdr, deg_sh.at[pl.ds(r0, WB_ROWS)])

    pltpu.sync_copy(onesr, ones_v)

    def idx_start(g, dst_b, semi):
      off = pl.multiple_of(g * G, G)
      pltpu.async_copy(edger.at[1, wid, pl.ds(off, G)], dst_b, semi)

    def idx_wait(g, dst_b, semi):
      off = pl.multiple_of(g * G, G)
      pltpu.make_async_copy(edger.at[1, wid, pl.ds(off, G)], dst_b, semi).wait()

    def process_group(dst_b):
      cps = [pltpu.async_copy(ones_v, deg_sh.at[dst_b.at[j]], sems, add=True)
             for j in range(G)]
      for cp in cps:
        cp.wait()

    idx_start(0, dst_b0, semi0)
    plsc.subcore_barrier()

    def pair(k, carry):
      g0 = 2 * k
      idx_start(g0 + 1, dst_b1, semi1)
      idx_wait(g0, dst_b0, semi0)
      process_group(dst_b0)

      @pl.when(k < NG // 2 - 1)
      def _prefetch():
        idx_start(g0 + 2, dst_b0, semi0)

      idx_wait(g0 + 1, dst_b1, semi1)
      process_group(dst_b1)
      return carry

    lax.fori_loop(0, NG // 2, pair, 0)
    plsc.subcore_barrier()

    @pl.when(s < WB_TILES)
    def _writeback():
      pltpu.sync_copy(deg_sh.at[pl.ds(r0, WB_ROWS)],
                      deg_out.at[c, pl.ds(r0, WB_ROWS)])

  scratch = [
      pltpu.VMEM((G, CH), jnp.int32),
      pltpu.VMEM((G, CH), jnp.int32),
      pltpu.VMEM((CH, DEGW), jnp.float32),
      pltpu.VMEM_SHARED((BN, DEGW), jnp.float32),
      pltpu.SemaphoreType.DMA,
      pltpu.SemaphoreType.DMA,
      pltpu.SemaphoreType.DMA,
  ]
  return pl.kernel(
      body, (jax.ShapeDtypeStruct((NC, BN, DEGW), jnp.float32),),
      mesh=mesh, scratch_types=tuple(scratch))


_sc_seg = _make_sc_seg()
_sc_deg = _make_sc_deg()

RB = 2000


def _msg_body(h_ref, w_ref, b_ref, o_ref):
  o_ref[...] = jnp.maximum(
      jnp.dot(h_ref[...], w_ref[...], preferred_element_type=jnp.float32)
      + b_ref[...], 0.0)


_msg = pl.pallas_call(
    _msg_body,
    grid=(BN // RB,),
    in_specs=[
        pl.BlockSpec((RB, DN), lambda i: (i, 0)),
        pl.BlockSpec((DN, DN), lambda i: (0, 0)),
        pl.BlockSpec((1, DN), lambda i: (0, 0)),
    ],
    out_specs=pl.BlockSpec((RB, DN), lambda i: (i, 0)),
    out_shape=jax.ShapeDtypeStruct((BN, DN), jnp.float32),
)


def _upd_body(h_ref, a_ref, d_ref, wu_ref, bu_ref, wm_ref, bm_ref,
              h1_ref, m2_ref):
  agg = a_ref[0] + a_ref[1]
  deg = d_ref[0, :, 0:1] + d_ref[1, :, 0:1]
  mask = jnp.where(deg > EPS, 1.0, 0.0)
  x = jnp.dot(h_ref[...], wu_ref[:DN], preferred_element_type=jnp.float32)
  x = x + jnp.dot(agg, wu_ref[DN:], preferred_element_type=jnp.float32)
  h1 = jnp.maximum(x + bu_ref[...], 0.0) * mask
  h1_ref[...] = h1
  m2_ref[...] = jnp.maximum(
      jnp.dot(h1, wm_ref[...], preferred_element_type=jnp.float32)
      + bm_ref[...], 0.0)


_upd = pl.pallas_call(
    _upd_body,
    grid=(BN // RB,),
    in_specs=[
        pl.BlockSpec((RB, DN), lambda i: (i, 0)),
        pl.BlockSpec((NC, RB, DN), lambda i: (0, i, 0)),
        pl.BlockSpec((NC, RB, DEGW), lambda i: (0, i, 0)),
        pl.BlockSpec((2 * DN, DN), lambda i: (0, 0)),
        pl.BlockSpec((1, DN), lambda i: (0, 0)),
        pl.BlockSpec((DN, DN), lambda i: (0, 0)),
        pl.BlockSpec((1, DN), lambda i: (0, 0)),
    ],
    out_specs=[
        pl.BlockSpec((RB, DN), lambda i: (i, 0)),
        pl.BlockSpec((RB, DN), lambda i: (i, 0)),
    ],
    out_shape=[
        jax.ShapeDtypeStruct((BN, DN), jnp.float32),
        jax.ShapeDtypeStruct((BN, DN), jnp.float32),
    ],
)


def _tail_body(h_ref, a_ref, d_ref, wu_ref, bu_ref, w1_ref, b1_ref,
               w2_ref, b2_ref, o_ref):
  agg = a_ref[0] + a_ref[1]
  deg = d_ref[0, :, 0:1] + d_ref[1, :, 0:1]
  mask = jnp.where(deg > EPS, 1.0, 0.0)
  x = jnp.dot(h_ref[...], wu_ref[:DN], preferred_element_type=jnp.float32)
  x = x + jnp.dot(agg, wu_ref[DN:], preferred_element_type=jnp.float32)
  h2 = jnp.maximum(x + bu_ref[...], 0.0) * mask
  gid = lax.broadcasted_iota(jnp.int32, (B, BN), 1) // N
  bid = lax.broadcasted_iota(jnp.int32, (B, BN), 0)
  sel = jnp.where(gid == bid, 1.0, 0.0)
  pooled = jnp.dot(sel, h2, preferred_element_type=jnp.float32)
  counts = jnp.dot(sel, mask, preferred_element_type=jnp.float32)
  combined = pooled / counts
  hidden = jnp.maximum(
      jnp.dot(combined, w1_ref[...], preferred_element_type=jnp.float32)
      + b1_ref[...], 0.0)
  o_ref[...] = jnp.dot(hidden, w2_ref[...],
                       preferred_element_type=jnp.float32) + b2_ref[...]


_tail = pl.pallas_call(
    _tail_body,
    out_shape=jax.ShapeDtypeStruct((B, DN), jnp.float32),
)


@jax.jit
def kernel(node_feats, edge_index, W_msg, b_msg, W_upd, b_upd, W_out1, b_out1,
           W_out2, b_out2):
  h0 = node_feats.reshape(BN, DIN)
  edges = edge_index.astype(jnp.int32).reshape(2, NW, NCH, CH)
  zeros = jnp.zeros((WB_ROWS, DN), jnp.float32)
  zerosd = jnp.zeros((WB_ROWS, DEGW), jnp.float32)
  ones = jnp.ones((CH, DEGW), jnp.float32)
  bm = b_msg.reshape(1, DN)
  bu = b_upd.reshape(1, DN)
  b1 = b_out1.reshape(1, H_OUT)
  W2p = jnp.pad(W_out2, ((0, 0), (0, DN - N_CLASSES)))
  b2p = jnp.pad(b_out2, (0, DN - N_CLASSES)).reshape(1, DN)

  m1 = _msg(h0, W_msg, bm)
  (degP,) = _sc_deg(edges, zerosd, ones)
  (aggP1,) = _sc_seg(m1, edges, zeros)
  h1, m2 = _upd(h0, aggP1, degP, W_upd, bu, W_msg, bm)
  (aggP2,) = _sc_seg(m2, edges, zeros)
  logits_p = _tail(h1, aggP2, degP, W_upd, bu, W_out1, b1, W2p, b2p)
  return logits_p[:, :N_CLASSES]


# speedup vs baseline: 12.0045x; 1.0253x over previous
import functools

import jax
import jax.numpy as jnp
from jax import lax
from jax.experimental import pallas as pl
from jax.experimental.pallas import tpu as pltpu
from jax.experimental.pallas import tpu_sc as plsc

EPS = 1e-05
B, N, DIN, DN = 8, 1250, 128, 128
E = 320000
H_OUT = 256
N_CLASSES = 10
BN = B * N

NC, NS = 2, 16
NW = NC * NS
EPT = E // NW
CH = 125
NCH = EPT // CH
WB_TILES = 10
WB_ROWS = BN // WB_TILES
DEGW = 16


G = 8
NG = NCH // G


def _make_sc_seg():
  mesh = plsc.VectorSubcoreMesh(core_axis_name="c", subcore_axis_name="s")

  def body(m_hbm, edger, zerosr, agg_out,
           src_b0, src_b1, dst_b0, dst_b1, rows0, rows1,
           agg_sh, semr0, semr1, semi0, semi1, sems0, sems1):
    c = lax.axis_index("c")
    s = lax.axis_index("s")
    wid = c * NS + s
    r0 = s * WB_ROWS
    rows = (rows0, rows1)
    semr = (semr0, semr1)

    @pl.when(s < WB_TILES)
    def _zero():
      pltpu.sync_copy(zerosr, agg_sh.at[pl.ds(r0, WB_ROWS)])

    def idx_start(g, src_b, dst_b, semi):
      off = pl.multiple_of(g * G, G)
      pltpu.async_copy(edger.at[0, wid, pl.ds(off, G)], src_b, semi)
      pltpu.async_copy(edger.at[1, wid, pl.ds(off, G)], dst_b, semi)

    def idx_wait(g, src_b, dst_b, semi):
      off = pl.multiple_of(g * G, G)
      pltpu.make_async_copy(edger.at[0, wid, pl.ds(off, G)], src_b, semi).wait()
      pltpu.make_async_copy(edger.at[1, wid, pl.ds(off, G)], dst_b, semi).wait()

    semsc = (sems0, sems1)

    def process_group(src_b, dst_b):
      gat = [None] * G
      scat = [None] * G
      for j in range(G):
        if j >= 2:
          scat[j - 2].wait()
        gat[j] = pltpu.async_copy(m_hbm.at[src_b.at[j]], rows[j % 2],
                                  semr[j % 2])
        if j >= 1:
          gat[j - 1].wait()
          scat[j - 1] = pltpu.async_copy(
              rows[(j - 1) % 2], agg_sh.at[dst_b.at[j - 1]],
              semsc[(j - 1) % 2], add=True)
      gat[G - 1].wait()
      scat[G - 1] = pltpu.async_copy(
          rows[(G - 1) % 2], agg_sh.at[dst_b.at[G - 1]],
          semsc[(G - 1) % 2], add=True)
      scat[G - 2].wait()
      scat[G - 1].wait()

    idx_start(0, src_b0, dst_b0, semi0)
    plsc.subcore_barrier()

    def pair(k, carry):
      g0 = 2 * k
      idx_start(g0 + 1, src_b1, dst_b1, semi1)
      idx_wait(g0, src_b0, dst_b0, semi0)
      process_group(src_b0, dst_b0)

      @pl.when(k < NG // 2 - 1)
      def _prefetch():
        idx_start(g0 + 2, src_b0, dst_b0, semi0)

      idx_wait(g0 + 1, src_b1, dst_b1, semi1)
      process_group(src_b1, dst_b1)
      return carry

    lax.fori_loop(0, NG // 2, pair, 0)
    plsc.subcore_barrier()

    @pl.when(s < WB_TILES)
    def _writeback():
      pltpu.sync_copy(agg_sh.at[pl.ds(r0, WB_ROWS)],
                      agg_out.at[c, pl.ds(r0, WB_ROWS)])

  scratch = [
      pltpu.VMEM((G, CH), jnp.int32),
      pltpu.VMEM((G, CH), jnp.int32),
      pltpu.VMEM((G, CH), jnp.int32),
      pltpu.VMEM((G, CH), jnp.int32),
      pltpu.VMEM((CH, DN), jnp.float32),
      pltpu.VMEM((CH, DN), jnp.float32),
      pltpu.VMEM_SHARED((BN, DN), jnp.float32),
      pltpu.SemaphoreType.DMA,
      pltpu.SemaphoreType.DMA,
      pltpu.SemaphoreType.DMA,
      pltpu.SemaphoreType.DMA,
      pltpu.SemaphoreType.DMA,
      pltpu.SemaphoreType.DMA,
  ]
  return pl.kernel(
      body, (jax.ShapeDtypeStruct((NC, BN, DN), jnp.float32),),
      mesh=mesh, scratch_types=tuple(scratch))


def _make_sc_deg():
  mesh = plsc.VectorSubcoreMesh(core_axis_name="c", subcore_axis_name="s")

  def body(edger, zerosdr, onesr, deg_out,
           dst_b0, dst_b1, ones_v, deg_sh, sems, semi0, semi1):
    c = lax.axis_index("c")
    s = lax.axis_index("s")
    wid = c * NS + s
    r0 = s * WB_ROWS

    @pl.when(s < WB_TILES)
    def _zero():
      pltpu.sync_copy(zerosdr, deg_sh.at[pl.ds(r0, WB_ROWS)])

    pltpu.sync_copy(onesr, ones_v)

    def idx_start(g, dst_b, semi):
      off = pl.multiple_of(g * G, G)
      pltpu.async_copy(edger.at[1, wid, pl.ds(off, G)], dst_b, semi)

    def idx_wait(g, dst_b, semi):
      off = pl.multiple_of(g * G, G)
      pltpu.make_async_copy(edger.at[1, wid, pl.ds(off, G)], dst_b, semi).wait()

    def process_group(dst_b):
      cps = [pltpu.async_copy(ones_v, deg_sh.at[dst_b.at[j]], sems, add=True)
             for j in range(G)]
      for cp in cps:
        cp.wait()

    idx_start(0, dst_b0, semi0)
    plsc.subcore_barrier()

    def pair(k, carry):
      g0 = 2 * k
      idx_start(g0 + 1, dst_b1, semi1)
      idx_wait(g0, dst_b0, semi0)
      process_group(dst_b0)

      @pl.when(k < NG // 2 - 1)
      def _prefetch():
        idx_start(g0 + 2, dst_b0, semi0)

      idx_wait(g0 + 1, dst_b1, semi1)
      process_group(dst_b1)
      return carry

    lax.fori_loop(0, NG // 2, pair, 0)
    plsc.subcore_barrier()

    @pl.when(s < WB_TILES)
    def _writeback():
      pltpu.sync_copy(deg_sh.at[pl.ds(r0, WB_ROWS)],
                      deg_out.at[c, pl.ds(r0, WB_ROWS)])

  scratch = [
      pltpu.VMEM((G, CH), jnp.int32),
      pltpu.VMEM((G, CH), jnp.int32),
      pltpu.VMEM((CH, DEGW), jnp.float32),
      pltpu.VMEM_SHARED((BN, DEGW), jnp.float32),
      pltpu.SemaphoreType.DMA,
      pltpu.SemaphoreType.DMA,
      pltpu.SemaphoreType.DMA,
  ]
  return pl.kernel(
      body, (jax.ShapeDtypeStruct((NC, BN, DEGW), jnp.float32),),
      mesh=mesh, scratch_types=tuple(scratch))


_sc_seg = _make_sc_seg()
_sc_deg = _make_sc_deg()

RB = 2000


def _msg_body(h_ref, w_ref, b_ref, o_ref):
  o_ref[...] = jnp.maximum(
      jnp.dot(h_ref[...], w_ref[...], preferred_element_type=jnp.float32)
      + b_ref[...], 0.0)


_msg = pl.pallas_call(
    _msg_body,
    grid=(BN // RB,),
    in_specs=[
        pl.BlockSpec((RB, DN), lambda i: (i, 0)),
        pl.BlockSpec((DN, DN), lambda i: (0, 0)),
        pl.BlockSpec((1, DN), lambda i: (0, 0)),
    ],
    out_specs=pl.BlockSpec((RB, DN), lambda i: (i, 0)),
    out_shape=jax.ShapeDtypeStruct((BN, DN), jnp.float32),
)


def _upd_body(h_ref, a_ref, d_ref, wu_ref, bu_ref, wm_ref, bm_ref,
              h1_ref, m2_ref):
  agg = a_ref[0] + a_ref[1]
  deg = d_ref[0, :, 0:1] + d_ref[1, :, 0:1]
  mask = jnp.where(deg > EPS, 1.0, 0.0)
  x = jnp.dot(h_ref[...], wu_ref[:DN], preferred_element_type=jnp.float32)
  x = x + jnp.dot(agg, wu_ref[DN:], preferred_element_type=jnp.float32)
  h1 = jnp.maximum(x + bu_ref[...], 0.0) * mask
  h1_ref[...] = h1
  m2_ref[...] = jnp.maximum(
      jnp.dot(h1, wm_ref[...], preferred_element_type=jnp.float32)
      + bm_ref[...], 0.0)


_upd = pl.pallas_call(
    _upd_body,
    grid=(BN // RB,),
    in_specs=[
        pl.BlockSpec((RB, DN), lambda i: (i, 0)),
        pl.BlockSpec((NC, RB, DN), lambda i: (0, i, 0)),
        pl.BlockSpec((NC, RB, DEGW), lambda i: (0, i, 0)),
        pl.BlockSpec((2 * DN, DN), lambda i: (0, 0)),
        pl.BlockSpec((1, DN), lambda i: (0, 0)),
        pl.BlockSpec((DN, DN), lambda i: (0, 0)),
        pl.BlockSpec((1, DN), lambda i: (0, 0)),
    ],
    out_specs=[
        pl.BlockSpec((RB, DN), lambda i: (i, 0)),
        pl.BlockSpec((RB, DN), lambda i: (i, 0)),
    ],
    out_shape=[
        jax.ShapeDtypeStruct((BN, DN), jnp.float32),
        jax.ShapeDtypeStruct((BN, DN), jnp.float32),
    ],
)


def _tail_body(h_ref, a_ref, d_ref, wu_ref, bu_ref, w1_ref, b1_ref,
               w2_ref, b2_ref, o_ref):
  agg = a_ref[0] + a_ref[1]
  deg = d_ref[0, :, 0:1] + d_ref[1, :, 0:1]
  mask = jnp.where(deg > EPS, 1.0, 0.0)
  x = jnp.dot(h_ref[...], wu_ref[:DN], preferred_element_type=jnp.float32)
  x = x + jnp.dot(agg, wu_ref[DN:], preferred_element_type=jnp.float32)
  h2 = jnp.maximum(x + bu_ref[...], 0.0) * mask
  gid = lax.broadcasted_iota(jnp.int32, (B, BN), 1) // N
  bid = lax.broadcasted_iota(jnp.int32, (B, BN), 0)
  sel = jnp.where(gid == bid, 1.0, 0.0)
  pooled = jnp.dot(sel, h2, preferred_element_type=jnp.float32)
  counts = jnp.dot(sel, mask, preferred_element_type=jnp.float32)
  combined = pooled / counts
  hidden = jnp.maximum(
      jnp.dot(combined, w1_ref[...], preferred_element_type=jnp.float32)
      + b1_ref[...], 0.0)
  o_ref[...] = jnp.dot(hidden, w2_ref[...],
                       preferred_element_type=jnp.float32) + b2_ref[...]


_tail = pl.pallas_call(
    _tail_body,
    out_shape=jax.ShapeDtypeStruct((B, DN), jnp.float32),
)


@jax.jit
def kernel(node_feats, edge_index, W_msg, b_msg, W_upd, b_upd, W_out1, b_out1,
           W_out2, b_out2):
  h0 = node_feats.reshape(BN, DIN)
  edges = edge_index.astype(jnp.int32).reshape(2, NW, NCH, CH)
  zeros = jnp.zeros((WB_ROWS, DN), jnp.float32)
  zerosd = jnp.zeros((WB_ROWS, DEGW), jnp.float32)
  ones = jnp.ones((CH, DEGW), jnp.float32)
  bm = b_msg.reshape(1, DN)
  bu = b_upd.reshape(1, DN)
  b1 = b_out1.reshape(1, H_OUT)
  W2p = jnp.pad(W_out2, ((0, 0), (0, DN - N_CLASSES)))
  b2p = jnp.pad(b_out2, (0, DN - N_CLASSES)).reshape(1, DN)

  m1 = _msg(h0, W_msg, bm)
  (degP,) = _sc_deg(edges, zerosd, ones)
  zeros = zeros + degP[0, :1, :1] * 0.0
  (aggP1,) = _sc_seg(m1, edges, zeros)
  h1, m2 = _upd(h0, aggP1, degP, W_upd, bu, W_msg, bm)
  (aggP2,) = _sc_seg(m2, edges, zeros)
  logits_p = _tail(h1, aggP2, degP, W_upd, bu, W_out1, b1, W2p, b2p)
  return logits_p[:, :N_CLASSES]
